# baseline (speedup 1.0000x reference)
import functools

import jax
import jax.numpy as jnp
import numpy as np
from jax import lax
from jax.experimental import pallas as pl
from jax.experimental.pallas import tpu as pltpu
from jax.experimental.pallas import tpu_sc as plsc

N = 10000
E = 160000
NF = 64
NODE_DIM = 256
AVG_NEIGH = 16.0
C1 = float(np.sqrt(3.0))

NCORE = 2
NSUB = 16
LANE = 16
HALF = NF // NCORE
ROWW = HALF * 4
RECW = 48
CHUNK = 80
NCH = E // CHUNK
CPT = NCH // NSUB
NBLK8 = N // 8
ZR = 52


def _node_lin_body(nf_ref, w_ref, outa_ref, outb_ref):
    lin = jnp.dot(nf_ref[...], w_ref[...], preferred_element_type=jnp.float32)
    outa_ref[...] = lin[:, :HALF]
    outb_ref[...] = lin[:, HALF:]


def _node_lin(node_feats, w1s):
    bn = 1000
    return pl.pallas_call(
        _node_lin_body,
        grid=(N // bn,),
        in_specs=[
            pl.BlockSpec((bn, NODE_DIM), lambda i: (i, 0)),
            pl.BlockSpec((NODE_DIM, NF), lambda i: (0, 0)),
        ],
        out_specs=[
            pl.BlockSpec((bn, HALF), lambda i: (i, 0)),
            pl.BlockSpec((bn, HALF), lambda i: (i, 0)),
        ],
        out_shape=[
            jax.ShapeDtypeStruct((N, HALF), jnp.float32),
            jax.ShapeDtypeStruct((N, HALF), jnp.float32),
        ],
    )(node_feats, w1s)


def _edge_body(ef_ref, len_ref, vec_ref, we1p_ref, we2_ref, rec_ref):
    be = ef_ref.shape[0]
    x = jnp.concatenate(
        [ef_ref[...], len_ref[...], jnp.zeros((be, 7), jnp.float32)], axis=1)
    r = jnp.dot(x, we1p_ref[...], preferred_element_type=jnp.float32)
    r = r * jax.nn.sigmoid(r)
    t = jnp.dot(r, we2_ref[...], preferred_element_type=jnp.float32)
    v = vec_ref[...]
    norm = jnp.sqrt(jnp.sum(v * v, axis=1, keepdims=True))
    shv = (C1 / (norm + 1e-9)) * v
    z13 = jnp.zeros((be, 13), jnp.float32)
    z45 = jnp.zeros((be, 45), jnp.float32)
    rec_ref[...] = jnp.concatenate(
        [t[:, :HALF], shv, z13, t[:, HALF:], shv, z45], axis=1)


def _edge_stage(edge_feats, lengths, vectors, we1p, we2):
    be = 8000
    return pl.pallas_call(
        _edge_body,
        grid=(E // be,),
        in_specs=[
            pl.BlockSpec((be, NF), lambda i: (i, 0)),
            pl.BlockSpec((be, 1), lambda i: (i, 0)),
            pl.BlockSpec((be, 3), lambda i: (i, 0)),
            pl.BlockSpec((NF + 8, NF), lambda i: (0, 0)),
            pl.BlockSpec((NF, NF), lambda i: (0, 0)),
        ],
        out_specs=pl.BlockSpec((be, 128), lambda i: (i, 0)),
        out_shape=jax.ShapeDtypeStruct((E, 128), jnp.float32),
    )(edge_feats, lengths, vectors, we1p, we2)


def _sc_body(sna_hbm, snb_hbm, rec_hbm, snd_hbm, rcv_hbm, out_hbm,
             sidx_v, ridx_v, rec_v, sj_v, rows_v, zbuf_v, agg_sh,
             semA, semB, semC):
    cid = lax.axis_index("c")
    sid = lax.axis_index("s")

    zv = jnp.zeros((LANE,), jnp.float32)

    def zrow(i, carry):
        for b in range(ROWW // LANE):
            zbuf_v[i, b * LANE:(b + 1) * LANE] = zv
        return carry

    lax.fori_loop(0, ZR, zrow, 0)

    nblk = NBLK8 // NSUB + jnp.where(sid < NBLK8 % NSUB, 1, 0)
    r0 = 8 * (NBLK8 // NSUB) * sid + 8 * jnp.minimum(sid, NBLK8 % NSUB)

    def zcp(t, carry):
        pltpu.sync_copy(zbuf_v, agg_sh.at[pl.ds(r0 + t * ZR, ZR)])
        return carry

    lax.fori_loop(0, (8 * (NBLK8 // NSUB)) // ZR, zcp, 0)

    @pl.when(sid < NBLK8 % NSUB)
    def _():
        pltpu.sync_copy(zbuf_v.at[pl.ds(0, 8)],
                        agg_sh.at[pl.ds(r0 + 8 * nblk - 8, 8)])

    plsc.subcore_barrier()

    col0 = cid * RECW

    def issueA(k, buf):
        e0 = (sid * CPT + k) * CHUNK
        pltpu.async_copy(snd_hbm.at[pl.ds(e0, CHUNK)], sidx_v.at[buf], semA)
        pltpu.async_copy(rcv_hbm.at[pl.ds(e0, CHUNK)],
                         ridx_v.at[lax.rem(k, 4)], semA)
        pltpu.async_copy(rec_hbm.at[pl.ds(e0, CHUNK), pl.ds(col0, RECW)],
                         rec_v.at[buf], semA)

    def waitA(buf, kk4):
        pltpu.make_async_copy(snd_hbm.at[pl.ds(0, CHUNK)],
                              sidx_v.at[buf], semA).wait()
        pltpu.make_async_copy(rcv_hbm.at[pl.ds(0, CHUNK)],
                              ridx_v.at[kk4], semA).wait()
        pltpu.make_async_copy(rec_hbm.at[pl.ds(0, CHUNK), pl.ds(0, RECW)],
                              rec_v.at[buf], semA).wait()

    def issueB(buf):
        @pl.when(cid == 0)
        def _():
            pltpu.async_copy(sna_hbm.at[sidx_v.at[buf]], sj_v.at[buf], semB)

        @pl.when(cid == 1)
        def _():
            pltpu.async_copy(snb_hbm.at[sidx_v.at[buf]], sj_v.at[buf], semB)

    def waitB(buf):
        pltpu.make_async_copy(sna_hbm.at[sidx_v.at[buf]],
                              sj_v.at[buf], semB).wait()

    def compute(buf):
        def edge(i, ecarry):
            t0 = rec_v[buf, i, 0:LANE]
            t1 = rec_v[buf, i, LANE:2 * LANE]
            shr = rec_v[buf, i, 2 * LANE:3 * LANE]
            s0 = sj_v[buf, i, 0:LANE]
            s1 = sj_v[buf, i, LANE:2 * LANE]
            u0 = s0 * t0
            u1 = s1 * t1
            sx = shr[0]
            sy = shr[1]
            sz = shr[2]
            rows_v[buf, i, 0:16] = u0
            rows_v[buf, i, 16:32] = u1
            rows_v[buf, i, 32:48] = u0 * sx
            rows_v[buf, i, 48:64] = u1 * sx
            rows_v[buf, i, 64:80] = u0 * sy
            rows_v[buf, i, 80:96] = u1 * sy
            rows_v[buf, i, 96:112] = u0 * sz
            rows_v[buf, i, 112:128] = u1 * sz
            return ecarry

        lax.fori_loop(0, CHUNK, edge, 0, unroll=16)

    issueA(0, 0)
    waitA(0, 0)
    issueB(0)
    issueA(1, 1)

    def step(k, carry):
        p = lax.rem(k, 2)
        q = 1 - p

        @pl.when(k + 1 < CPT)
        def _():
            waitA(q, lax.rem(k + 1, 4))
            issueB(q)

        waitB(p)

        @pl.when(k >= 2)
        def _():
            pltpu.make_async_copy(out_hbm.at[pl.ds(0, CHUNK)],
                                  rows_v.at[p], semC).wait()

        compute(p)
        pltpu.async_copy(rows_v.at[p], agg_sh.at[ridx_v.at[lax.rem(k, 4)]],
                         semC, add=True)

        @pl.when(k + 2 < CPT)
        def _():
            issueA(k + 2, p)

        return carry

    lax.fori_loop(0, CPT, step, 0)
    pltpu.make_async_copy(out_hbm.at[pl.ds(0, CHUNK)],
                          rows_v.at[lax.rem(CPT - 2, 2)], semC).wait()
    pltpu.make_async_copy(out_hbm.at[pl.ds(0, CHUNK)],
                          rows_v.at[lax.rem(CPT - 1, 2)], semC).wait()

    @pl.when(sid < NCH - CPT * NSUB)
    def _():
        e0 = (CPT * NSUB + sid) * CHUNK
        pltpu.async_copy(snd_hbm.at[pl.ds(e0, CHUNK)], sidx_v.at[0], semA)
        pltpu.async_copy(rcv_hbm.at[pl.ds(e0, CHUNK)], ridx_v.at[0], semA)
        pltpu.async_copy(rec_hbm.at[pl.ds(e0, CHUNK), pl.ds(col0, RECW)],
                         rec_v.at[0], semA)
        waitA(0, lax.rem(CPT * NSUB + sid, 1) * 0)
        issueB(0)
        waitB(0)
        compute(0)
        pltpu.sync_copy(rows_v.at[0], agg_sh.at[ridx_v.at[0]], add=True)

    plsc.subcore_barrier()

    def ocp(t, carry):
        pltpu.sync_copy(agg_sh.at[pl.ds(r0 + t * ZR, ZR)],
                        out_hbm.at[pl.ds(cid * N + r0 + t * ZR, ZR)])
        return carry

    lax.fori_loop(0, (8 * (NBLK8 // NSUB)) // ZR, ocp, 0)

    @pl.when(sid < NBLK8 % NSUB)
    def _():
        rt = r0 + 8 * nblk - 8
        pltpu.sync_copy(agg_sh.at[pl.ds(rt, 8)],
                        out_hbm.at[pl.ds(cid * N + rt, 8)])


def _sc_scatter(sna, snb, rec, senders, receivers):
    mesh = plsc.VectorSubcoreMesh(core_axis_name="c", subcore_axis_name="s",
                                  num_cores=NCORE, num_subcores=NSUB)
    f = functools.partial(
        pl.kernel,
        out_type=jax.ShapeDtypeStruct((NCORE * N, ROWW), jnp.float32),
        mesh=mesh,
        scratch_types=[
            pltpu.VMEM((2, CHUNK), jnp.int32),
            pltpu.VMEM((4, CHUNK), jnp.int32),
            pltpu.VMEM((2, CHUNK, RECW), jnp.float32),
            pltpu.VMEM((2, CHUNK, HALF), jnp.float32),
            pltpu.VMEM((2, CHUNK, ROWW), jnp.float32),
            pltpu.VMEM((ZR, ROWW), jnp.float32),
            pltpu.VMEM_SHARED((N, ROWW), jnp.float32),
            pltpu.SemaphoreType.DMA,
            pltpu.SemaphoreType.DMA,
            pltpu.SemaphoreType.DMA,
        ],
        compiler_params=pltpu.CompilerParams(use_tc_tiling_on_sc=False),
    )(_sc_body)
    return f(sna, snb, rec, senders, receivers)


def _post_body(a_ref, b_ref, w2s_ref, wp1_ref, wp2_ref, wp3_ref, wv_ref,
               wg_ref, wr1_ref, wr2_ref, wrv_ref,
               os_ref, ov_ref, sout_ref, v0_ref, v1_ref, v2_ref):
    inv = 1.0 / AVG_NEIGH
    cat = jnp.concatenate([a_ref[...], b_ref[...]], axis=1) * inv
    m_all = jnp.dot(cat, w2s_ref[...], preferred_element_type=jnp.float32)
    s = m_all[:, 0:64]
    v0 = m_all[:, 64:128]
    v1 = m_all[:, 128:192]
    v2 = m_all[:, 192:256]
    s2 = s * s
    s_out = (jnp.dot(s, wp1_ref[...], preferred_element_type=jnp.float32)
             + jnp.dot(s2, wp2_ref[...], preferred_element_type=jnp.float32)
             + jnp.dot(s2 * s, wp3_ref[...], preferred_element_type=jnp.float32))
    g = jnp.dot(s, wg_ref[...], preferred_element_type=jnp.float32)
    gate = g * jax.nn.sigmoid(g)
    vo0 = jnp.dot(v0, wv_ref[...], preferred_element_type=jnp.float32) * gate
    vo1 = jnp.dot(v1, wv_ref[...], preferred_element_type=jnp.float32) * gate
    vo2 = jnp.dot(v2, wv_ref[...], preferred_element_type=jnp.float32) * gate
    hpre = jnp.dot(s_out, wr1_ref[...], preferred_element_type=jnp.float32)
    h = hpre * jax.nn.sigmoid(hpre)
    os_ref[...] = jnp.dot(h, wr2_ref[...], preferred_element_type=jnp.float32)
    vgate = jnp.dot(h, wrv_ref[...], preferred_element_type=jnp.float32)
    ov_ref[...] = jnp.concatenate(
        [jnp.sum(vo0 * vgate, axis=1, keepdims=True),
         jnp.sum(vo1 * vgate, axis=1, keepdims=True),
         jnp.sum(vo2 * vgate, axis=1, keepdims=True)], axis=1)
    sout_ref[...] = s_out
    v0_ref[...] = vo0
    v1_ref[...] = vo1
    v2_ref[...] = vo2


def _post_stage(agg2, w2s, Wp1, Wp2, Wp3, Wv, Wg, Wr1, Wr2, Wrv):
    bn = 1000
    nblk = N // bn
    wspec = lambda shape: pl.BlockSpec(shape, lambda i: (0, 0))
    return pl.pallas_call(
        _post_body,
        grid=(nblk,),
        in_specs=[
            pl.BlockSpec((bn, ROWW), lambda i: (i, 0)),
            pl.BlockSpec((bn, ROWW), lambda i: (i + nblk, 0)),
            wspec((2 * ROWW, 256)),
            wspec((NF, NF)), wspec((NF, NF)), wspec((NF, NF)),
            wspec((NF, NF)), wspec((NF, NF)),
            wspec((NF, 16)), wspec((16, NF)), wspec((16, NF)),
        ],
        out_specs=[
            pl.BlockSpec((bn, NF), lambda i: (i, 0)),
            pl.BlockSpec((bn, 3), lambda i: (i, 0)),
            pl.BlockSpec((bn, NF), lambda i: (i, 0)),
            pl.BlockSpec((bn, NF), lambda i: (i, 0)),
            pl.BlockSpec((bn, NF), lambda i: (i, 0)),
            pl.BlockSpec((bn, NF), lambda i: (i, 0)),
        ],
        out_shape=[
            jax.ShapeDtypeStruct((N, NF), jnp.float32),
            jax.ShapeDtypeStruct((N, 3), jnp.float32),
            jax.ShapeDtypeStruct((N, NF), jnp.float32),
            jax.ShapeDtypeStruct((N, NF), jnp.float32),
            jax.ShapeDtypeStruct((N, NF), jnp.float32),
            jax.ShapeDtypeStruct((N, NF), jnp.float32),
        ],
    )(agg2, agg2, w2s, Wp1, Wp2, Wp3, Wv, Wg, Wr1, Wr2, Wrv)


def kernel(vectors, lengths, node_feats, edge_feats, edge_index,
           W1, We1, We2, W2, Wp1, Wp2, Wp3, Wv, Wg, Wr1, Wr2, Wrv):
    senders = edge_index[0]
    receivers = edge_index[1]

    sna, snb = _node_lin(node_feats, W1[:, :NF])
    we1p = jnp.pad(We1, ((0, 7), (0, 0)))
    rec = _edge_stage(edge_feats, lengths, vectors, we1p, We2)

    agg2 = _sc_scatter(sna, snb, rec, senders, receivers)

    eye4 = jnp.eye(4, dtype=W2.dtype)
    w2s = jnp.concatenate([jnp.kron(eye4, W2[:HALF]),
                           jnp.kron(eye4, W2[HALF:])], axis=0)

    out_scalars, out_vec, s_out, vo0, vo1, vo2 = _post_stage(
        agg2, w2s, Wp1, Wp2, Wp3, Wv, Wg, Wr1, Wr2, Wrv)

    vflat = jnp.stack([vo0, vo1, vo2], axis=2).reshape(N, 3 * NF)
    node_feats_out = jnp.concatenate([s_out, vflat], axis=1)
    return (out_scalars, out_vec, node_feats_out)


# speedup vs baseline: 1.0006x; 1.0006x over previous
import functools

import jax
import jax.numpy as jnp
import numpy as np
from jax import lax
from jax.experimental import pallas as pl
from jax.experimental.pallas import tpu as pltpu
from jax.experimental.pallas import tpu_sc as plsc

N = 10000
E = 160000
NF = 64
NODE_DIM = 256
AVG_NEIGH = 16.0
C1 = float(np.sqrt(3.0))

NCORE = 2
NSUB = 16
LANE = 16
HALF = NF // NCORE
ROWW = HALF * 4
RECW = 48
CHUNK = 80
NCH = E // CHUNK
CPT = NCH // NSUB
NBLK8 = N // 8
ZR = 52


def _node_lin_body(nf_ref, w_ref, outa_ref, outb_ref):
    lin = jnp.dot(nf_ref[...], w_ref[...], preferred_element_type=jnp.float32)
    outa_ref[...] = lin[:, :HALF]
    outb_ref[...] = lin[:, HALF:]


def _node_lin(node_feats, w1s):
    bn = 1000
    return pl.pallas_call(
        _node_lin_body,
        grid=(N // bn,),
        in_specs=[
            pl.BlockSpec((bn, NODE_DIM), lambda i: (i, 0)),
            pl.BlockSpec((NODE_DIM, NF), lambda i: (0, 0)),
        ],
        out_specs=[
            pl.BlockSpec((bn, HALF), lambda i: (i, 0)),
            pl.BlockSpec((bn, HALF), lambda i: (i, 0)),
        ],
        out_shape=[
            jax.ShapeDtypeStruct((N, HALF), jnp.float32),
            jax.ShapeDtypeStruct((N, HALF), jnp.float32),
        ],
    )(node_feats, w1s)


def _edge_body(ef_ref, len_ref, vec_ref, we1p_ref, we2_ref, rec_ref):
    be = ef_ref.shape[0]
    x = jnp.concatenate(
        [ef_ref[...], len_ref[...], jnp.zeros((be, 7), jnp.float32)], axis=1)
    r = jnp.dot(x, we1p_ref[...], preferred_element_type=jnp.float32)
    r = r * jax.nn.sigmoid(r)
    t = jnp.dot(r, we2_ref[...], preferred_element_type=jnp.float32)
    v = vec_ref[...]
    norm = jnp.sqrt(jnp.sum(v * v, axis=1, keepdims=True))
    shv = (C1 / (norm + 1e-9)) * v
    z13 = jnp.zeros((be, 13), jnp.float32)
    z45 = jnp.zeros((be, 45), jnp.float32)
    rec_ref[...] = jnp.concatenate(
        [t[:, :HALF], shv, z13, t[:, HALF:], shv, z45], axis=1)


def _edge_stage(edge_feats, lengths, vectors, we1p, we2):
    be = 8000
    return pl.pallas_call(
        _edge_body,
        grid=(E // be,),
        in_specs=[
            pl.BlockSpec((be, NF), lambda i: (i, 0)),
            pl.BlockSpec((be, 1), lambda i: (i, 0)),
            pl.BlockSpec((be, 3), lambda i: (i, 0)),
            pl.BlockSpec((NF + 8, NF), lambda i: (0, 0)),
            pl.BlockSpec((NF, NF), lambda i: (0, 0)),
        ],
        out_specs=pl.BlockSpec((be, 128), lambda i: (i, 0)),
        out_shape=jax.ShapeDtypeStruct((E, 128), jnp.float32),
    )(edge_feats, lengths, vectors, we1p, we2)


def _sc_body(sna_hbm, snb_hbm, rec_hbm, snd_hbm, rcv_hbm, out_hbm,
             sidx_v, ridx_v, rec_v, sj_v, rows_v, zbuf_v, agg_sh,
             semA, semB, semC):
    cid = lax.axis_index("c")
    sid = lax.axis_index("s")

    zv = jnp.zeros((LANE,), jnp.float32)

    def zrow(i, carry):
        for b in range(ROWW // LANE):
            zbuf_v[i, b * LANE:(b + 1) * LANE] = zv
        return carry

    lax.fori_loop(0, ZR, zrow, 0)

    nblk = NBLK8 // NSUB + jnp.where(sid < NBLK8 % NSUB, 1, 0)
    r0 = 8 * (NBLK8 // NSUB) * sid + 8 * jnp.minimum(sid, NBLK8 % NSUB)

    def zcp(t, carry):
        pltpu.sync_copy(zbuf_v, agg_sh.at[pl.ds(r0 + t * ZR, ZR)])
        return carry

    lax.fori_loop(0, (8 * (NBLK8 // NSUB)) // ZR, zcp, 0)

    @pl.when(sid < NBLK8 % NSUB)
    def _():
        pltpu.sync_copy(zbuf_v.at[pl.ds(0, 8)],
                        agg_sh.at[pl.ds(r0 + 8 * nblk - 8, 8)])

    plsc.subcore_barrier()

    col0 = cid * RECW

    def issueA(k, buf):
        e0 = (sid * CPT + k) * CHUNK
        pltpu.async_copy(snd_hbm.at[pl.ds(e0, CHUNK)], sidx_v.at[buf], semA)
        pltpu.async_copy(rcv_hbm.at[pl.ds(e0, CHUNK)],
                         ridx_v.at[lax.rem(k, 4)], semA)
        pltpu.async_copy(rec_hbm.at[pl.ds(e0, CHUNK), pl.ds(col0, RECW)],
                         rec_v.at[buf], semA)

    def waitA(buf, kk4):
        pltpu.make_async_copy(snd_hbm.at[pl.ds(0, CHUNK)],
                              sidx_v.at[buf], semA).wait()
        pltpu.make_async_copy(rcv_hbm.at[pl.ds(0, CHUNK)],
                              ridx_v.at[kk4], semA).wait()
        pltpu.make_async_copy(rec_hbm.at[pl.ds(0, CHUNK), pl.ds(0, RECW)],
                              rec_v.at[buf], semA).wait()

    def issueB(buf):
        @pl.when(cid == 0)
        def _():
            pltpu.async_copy(sna_hbm.at[sidx_v.at[buf]], sj_v.at[buf], semB)

        @pl.when(cid == 1)
        def _():
            pltpu.async_copy(snb_hbm.at[sidx_v.at[buf]], sj_v.at[buf], semB)

    def waitB(buf):
        pltpu.make_async_copy(sna_hbm.at[sidx_v.at[buf]],
                              sj_v.at[buf], semB).wait()

    def compute(buf):
        def edge(i, ecarry):
            t0 = rec_v[buf, i, 0:LANE]
            t1 = rec_v[buf, i, LANE:2 * LANE]
            shr = rec_v[buf, i, 2 * LANE:3 * LANE]
            s0 = sj_v[buf, i, 0:LANE]
            s1 = sj_v[buf, i, LANE:2 * LANE]
            u0 = s0 * t0
            u1 = s1 * t1
            sx = shr[0]
            sy = shr[1]
            sz = shr[2]
            rows_v[buf, i, 0:16] = u0
            rows_v[buf, i, 16:32] = u1
            rows_v[buf, i, 32:48] = u0 * sx
            rows_v[buf, i, 48:64] = u1 * sx
            rows_v[buf, i, 64:80] = u0 * sy
            rows_v[buf, i, 80:96] = u1 * sy
            rows_v[buf, i, 96:112] = u0 * sz
            rows_v[buf, i, 112:128] = u1 * sz
            return ecarry

        lax.fori_loop(0, CHUNK, edge, 0, unroll=16)

    issueA(0, 0)
    waitA(0, 0)
    issueB(0)
    issueA(1, 1)

    def step(k, carry):
        p = lax.rem(k, 2)
        q = 1 - p

        @pl.when(k + 1 < CPT)
        def _():
            waitA(q, lax.rem(k + 1, 4))
            issueB(q)

        waitB(p)

        @pl.when(k >= 2)
        def _():
            pltpu.make_async_copy(out_hbm.at[pl.ds(0, CHUNK)],
                                  rows_v.at[p], semC).wait()

        compute(p)
        pltpu.async_copy(rows_v.at[p], agg_sh.at[ridx_v.at[lax.rem(k, 4)]],
                         semC, add=True)

        @pl.when(k + 2 < CPT)
        def _():
            issueA(k + 2, p)

        return carry

    lax.fori_loop(0, CPT, step, 0)
    pltpu.make_async_copy(out_hbm.at[pl.ds(0, CHUNK)],
                          rows_v.at[lax.rem(CPT - 2, 2)], semC).wait()
    pltpu.make_async_copy(out_hbm.at[pl.ds(0, CHUNK)],
                          rows_v.at[lax.rem(CPT - 1, 2)], semC).wait()

    plsc.subcore_barrier()

    def ocp(t, carry):
        pltpu.sync_copy(agg_sh.at[pl.ds(r0 + t * ZR, ZR)],
                        out_hbm.at[pl.ds(cid * N + r0 + t * ZR, ZR)])
        return carry

    lax.fori_loop(0, (8 * (NBLK8 // NSUB)) // ZR, ocp, 0)

    @pl.when(sid < NBLK8 % NSUB)
    def _():
        rt = r0 + 8 * nblk - 8
        pltpu.sync_copy(agg_sh.at[pl.ds(rt, 8)],
                        out_hbm.at[pl.ds(cid * N + rt, 8)])


def _sc_scatter(sna, snb, rec, senders, receivers):
    mesh = plsc.VectorSubcoreMesh(core_axis_name="c", subcore_axis_name="s",
                                  num_cores=NCORE, num_subcores=NSUB)
    f = functools.partial(
        pl.kernel,
        out_type=jax.ShapeDtypeStruct((NCORE * N, ROWW), jnp.float32),
        mesh=mesh,
        scratch_types=[
            pltpu.VMEM((2, CHUNK), jnp.int32),
            pltpu.VMEM((4, CHUNK), jnp.int32),
            pltpu.VMEM((2, CHUNK, RECW), jnp.float32),
            pltpu.VMEM((2, CHUNK, HALF), jnp.float32),
            pltpu.VMEM((2, CHUNK, ROWW), jnp.float32),
            pltpu.VMEM((ZR, ROWW), jnp.float32),
            pltpu.VMEM_SHARED((N, ROWW), jnp.float32),
            pltpu.SemaphoreType.DMA,
            pltpu.SemaphoreType.DMA,
            pltpu.SemaphoreType.DMA,
        ],
        compiler_params=pltpu.CompilerParams(use_tc_tiling_on_sc=False),
    )(_sc_body)
    return f(sna, snb, rec, senders, receivers)


def _post_body(a_ref, b_ref, w2s_ref, wp1_ref, wp2_ref, wp3_ref, wv_ref,
               wg_ref, wr1_ref, wr2_ref, wrv_ref,
               os_ref, ov_ref, sout_ref, v0_ref, v1_ref, v2_ref):
    inv = 1.0 / AVG_NEIGH
    cat = jnp.concatenate([a_ref[...], b_ref[...]], axis=1) * inv
    m_all = jnp.dot(cat, w2s_ref[...], preferred_element_type=jnp.float32)
    s = m_all[:, 0:64]
    v0 = m_all[:, 64:128]
    v1 = m_all[:, 128:192]
    v2 = m_all[:, 192:256]
    s2 = s * s
    s_out = (jnp.dot(s, wp1_ref[...], preferred_element_type=jnp.float32)
             + jnp.dot(s2, wp2_ref[...], preferred_element_type=jnp.float32)
             + jnp.dot(s2 * s, wp3_ref[...], preferred_element_type=jnp.float32))
    g = jnp.dot(s, wg_ref[...], preferred_element_type=jnp.float32)
    gate = g * jax.nn.sigmoid(g)
    vo0 = jnp.dot(v0, wv_ref[...], preferred_element_type=jnp.float32) * gate
    vo1 = jnp.dot(v1, wv_ref[...], preferred_element_type=jnp.float32) * gate
    vo2 = jnp.dot(v2, wv_ref[...], preferred_element_type=jnp.float32) * gate
    hpre = jnp.dot(s_out, wr1_ref[...], preferred_element_type=jnp.float32)
    h = hpre * jax.nn.sigmoid(hpre)
    os_ref[...] = jnp.dot(h, wr2_ref[...], preferred_element_type=jnp.float32)
    vgate = jnp.dot(h, wrv_ref[...], preferred_element_type=jnp.float32)
    ov_ref[...] = jnp.concatenate(
        [jnp.sum(vo0 * vgate, axis=1, keepdims=True),
         jnp.sum(vo1 * vgate, axis=1, keepdims=True),
         jnp.sum(vo2 * vgate, axis=1, keepdims=True)], axis=1)
    sout_ref[...] = s_out
    v0_ref[...] = vo0
    v1_ref[...] = vo1
    v2_ref[...] = vo2


def _post_stage(agg2, w2s, Wp1, Wp2, Wp3, Wv, Wg, Wr1, Wr2, Wrv):
    bn = 1000
    nblk = N // bn
    wspec = lambda shape: pl.BlockSpec(shape, lambda i: (0, 0))
    return pl.pallas_call(
        _post_body,
        grid=(nblk,),
        in_specs=[
            pl.BlockSpec((bn, ROWW), lambda i: (i, 0)),
            pl.BlockSpec((bn, ROWW), lambda i: (i + nblk, 0)),
            wspec((2 * ROWW, 256)),
            wspec((NF, NF)), wspec((NF, NF)), wspec((NF, NF)),
            wspec((NF, NF)), wspec((NF, NF)),
            wspec((NF, 16)), wspec((16, NF)), wspec((16, NF)),
        ],
        out_specs=[
            pl.BlockSpec((bn, NF), lambda i: (i, 0)),
            pl.BlockSpec((bn, 3), lambda i: (i, 0)),
            pl.BlockSpec((bn, NF), lambda i: (i, 0)),
            pl.BlockSpec((bn, NF), lambda i: (i, 0)),
            pl.BlockSpec((bn, NF), lambda i: (i, 0)),
            pl.BlockSpec((bn, NF), lambda i: (i, 0)),
        ],
        out_shape=[
            jax.ShapeDtypeStruct((N, NF), jnp.float32),
            jax.ShapeDtypeStruct((N, 3), jnp.float32),
            jax.ShapeDtypeStruct((N, NF), jnp.float32),
            jax.ShapeDtypeStruct((N, NF), jnp.float32),
            jax.ShapeDtypeStruct((N, NF), jnp.float32),
            jax.ShapeDtypeStruct((N, NF), jnp.float32),
        ],
    )(agg2, agg2, w2s, Wp1, Wp2, Wp3, Wv, Wg, Wr1, Wr2, Wrv)


def kernel(vectors, lengths, node_feats, edge_feats, edge_index,
           W1, We1, We2, W2, Wp1, Wp2, Wp3, Wv, Wg, Wr1, Wr2, Wrv):
    senders = edge_index[0]
    receivers = edge_index[1]

    sna, snb = _node_lin(node_feats, W1[:, :NF])
    we1p = jnp.pad(We1, ((0, 7), (0, 0)))
    rec = _edge_stage(edge_feats, lengths, vectors, we1p, We2)

    agg2 = _sc_scatter(sna, snb, rec, senders, receivers)

    eye4 = jnp.eye(4, dtype=W2.dtype)
    w2s = jnp.concatenate([jnp.kron(eye4, W2[:HALF]),
                           jnp.kron(eye4, W2[HALF:])], axis=0)

    out_scalars, out_vec, s_out, vo0, vo1, vo2 = _post_stage(
        agg2, w2s, Wp1, Wp2, Wp3, Wv, Wg, Wr1, Wr2, Wrv)

    vflat = jnp.stack([vo0, vo1, vo2], axis=2).reshape(N, 3 * NF)
    node_feats_out = jnp.concatenate([s_out, vflat], axis=1)
    return (out_scalars, out_vec, node_feats_out)


# speedup vs baseline: 1.0565x; 1.0559x over previous
import functools

import jax
import jax.numpy as jnp
import numpy as np
from jax import lax
from jax.experimental import pallas as pl
from jax.experimental.pallas import tpu as pltpu
from jax.experimental.pallas import tpu_sc as plsc

N = 10000
E = 160000
NF = 64
NODE_DIM = 256
AVG_NEIGH = 16.0
C1 = float(np.sqrt(3.0))

NCORE = 2
NSUB = 16
LANE = 16
HALF = NF // NCORE
ROWW = HALF * 4
RECW = 48
CHUNK = 80
NCH = E // CHUNK
CPT = NCH // NSUB
NBLK8 = N // 8
ZR = 52


def _node_lin_body(nf_ref, w_ref, outa_ref, outb_ref):
    lin = jnp.dot(nf_ref[...], w_ref[...], preferred_element_type=jnp.float32)
    outa_ref[...] = lin[:, :HALF]
    outb_ref[...] = lin[:, HALF:]


def _node_lin(node_feats, w1s):
    bn = 1000
    return pl.pallas_call(
        _node_lin_body,
        grid=(N // bn,),
        in_specs=[
            pl.BlockSpec((bn, NODE_DIM), lambda i: (i, 0)),
            pl.BlockSpec((NODE_DIM, NF), lambda i: (0, 0)),
        ],
        out_specs=[
            pl.BlockSpec((bn, HALF), lambda i: (i, 0)),
            pl.BlockSpec((bn, HALF), lambda i: (i, 0)),
        ],
        out_shape=[
            jax.ShapeDtypeStruct((N, HALF), jnp.float32),
            jax.ShapeDtypeStruct((N, HALF), jnp.float32),
        ],
    )(node_feats, w1s)


def _edge_body(ef_ref, len_ref, vec_ref, we1p_ref, we2_ref, rec_ref):
    be = ef_ref.shape[0]
    x = jnp.concatenate(
        [ef_ref[...], len_ref[...], jnp.zeros((be, 7), jnp.float32)], axis=1)
    r = jnp.dot(x, we1p_ref[...], preferred_element_type=jnp.float32)
    r = r * jax.nn.sigmoid(r)
    t = jnp.dot(r, we2_ref[...], preferred_element_type=jnp.float32)
    v = vec_ref[...]
    norm = jnp.sqrt(jnp.sum(v * v, axis=1, keepdims=True))
    shv = (C1 / (norm + 1e-9)) * v
    z13 = jnp.zeros((be, 13), jnp.float32)
    z45 = jnp.zeros((be, 45), jnp.float32)
    rec_ref[...] = jnp.concatenate(
        [t[:, :HALF], shv, z13, t[:, HALF:], shv, z45], axis=1)


def _edge_stage(edge_feats, lengths, vectors, we1p, we2):
    be = 8000
    return pl.pallas_call(
        _edge_body,
        grid=(E // be,),
        in_specs=[
            pl.BlockSpec((be, NF), lambda i: (i, 0)),
            pl.BlockSpec((be, 1), lambda i: (i, 0)),
            pl.BlockSpec((be, 3), lambda i: (i, 0)),
            pl.BlockSpec((NF + 8, NF), lambda i: (0, 0)),
            pl.BlockSpec((NF, NF), lambda i: (0, 0)),
        ],
        out_specs=pl.BlockSpec((be, 128), lambda i: (i, 0)),
        out_shape=jax.ShapeDtypeStruct((E, 128), jnp.float32),
    )(edge_feats, lengths, vectors, we1p, we2)


def _sc_body(sna_hbm, snb_hbm, rec_hbm, snd_hbm, rcv_hbm, out_hbm,
             sidx_v, ridx_v, rec_v, sj_v, rows_v, zbuf_v, agg_sh,
             semA, semB, semC):
    cid = lax.axis_index("c")
    sid = lax.axis_index("s")

    zv = jnp.zeros((LANE,), jnp.float32)

    def zrow(i, carry):
        for b in range(ROWW // LANE):
            zbuf_v[i, b * LANE:(b + 1) * LANE] = zv
        return carry

    lax.fori_loop(0, ZR, zrow, 0)

    nblk = NBLK8 // NSUB + jnp.where(sid < NBLK8 % NSUB, 1, 0)
    r0 = 8 * (NBLK8 // NSUB) * sid + 8 * jnp.minimum(sid, NBLK8 % NSUB)

    def zcp(t, carry):
        pltpu.sync_copy(zbuf_v, agg_sh.at[pl.ds(r0 + t * ZR, ZR)])
        return carry

    lax.fori_loop(0, (8 * (NBLK8 // NSUB)) // ZR, zcp, 0)

    @pl.when(sid < NBLK8 % NSUB)
    def _():
        pltpu.sync_copy(zbuf_v.at[pl.ds(0, 8)],
                        agg_sh.at[pl.ds(r0 + 8 * nblk - 8, 8)])

    plsc.subcore_barrier()

    col0 = cid * RECW

    def issueA(k, buf):
        e0 = (sid * CPT + k) * CHUNK
        pltpu.async_copy(snd_hbm.at[pl.ds(e0, CHUNK)], sidx_v.at[buf], semA)
        pltpu.async_copy(rcv_hbm.at[pl.ds(e0, CHUNK)],
                         ridx_v.at[lax.rem(k, 4)], semA)
        pltpu.async_copy(rec_hbm.at[pl.ds(e0, CHUNK), pl.ds(col0, RECW)],
                         rec_v.at[buf], semA)

    def waitA(buf, kk4):
        pltpu.make_async_copy(snd_hbm.at[pl.ds(0, CHUNK)],
                              sidx_v.at[buf], semA).wait()
        pltpu.make_async_copy(rcv_hbm.at[pl.ds(0, CHUNK)],
                              ridx_v.at[kk4], semA).wait()
        pltpu.make_async_copy(rec_hbm.at[pl.ds(0, CHUNK), pl.ds(0, RECW)],
                              rec_v.at[buf], semA).wait()

    def issueB(buf):
        @pl.when(cid == 0)
        def _():
            pltpu.async_copy(sna_hbm.at[sidx_v.at[buf]], sj_v.at[buf], semB)

        @pl.when(cid == 1)
        def _():
            pltpu.async_copy(snb_hbm.at[sidx_v.at[buf]], sj_v.at[buf], semB)

    def waitB(buf):
        pltpu.make_async_copy(sna_hbm.at[sidx_v.at[buf]],
                              sj_v.at[buf], semB).wait()

    def compute(buf):
        def edge(i, ecarry):
            t0 = rec_v[buf, i, 0:LANE]
            t1 = rec_v[buf, i, LANE:2 * LANE]
            shr = rec_v[buf, i, 2 * LANE:3 * LANE]
            s0 = sj_v[buf, i, 0:LANE]
            s1 = sj_v[buf, i, LANE:2 * LANE]
            u0 = s0 * t0
            u1 = s1 * t1
            sx = shr[0]
            sy = shr[1]
            sz = shr[2]
            rows_v[buf, i, 0:16] = u0
            rows_v[buf, i, 16:32] = u1
            rows_v[buf, i, 32:48] = u0 * sx
            rows_v[buf, i, 48:64] = u1 * sx
            rows_v[buf, i, 64:80] = u0 * sy
            rows_v[buf, i, 80:96] = u1 * sy
            rows_v[buf, i, 96:112] = u0 * sz
            rows_v[buf, i, 112:128] = u1 * sz
            return ecarry

        lax.fori_loop(0, CHUNK, edge, 0, unroll=16)

    issueA(0, 0)
    waitA(0, 0)
    issueB(0)
    issueA(1, 1)

    def step(k, carry):
        p = lax.rem(k, 2)
        q = 1 - p

        @pl.when(k + 1 < CPT)
        def _():
            waitA(q, lax.rem(k + 1, 4))
            issueB(q)

        waitB(p)

        @pl.when(k >= 2)
        def _():
            pltpu.make_async_copy(out_hbm.at[pl.ds(0, CHUNK)],
                                  rows_v.at[p], semC).wait()

        compute(p)
        pltpu.async_copy(rows_v.at[p], agg_sh.at[ridx_v.at[lax.rem(k, 4)]],
                         semC, add=True)

        @pl.when(k + 2 < CPT)
        def _():
            issueA(k + 2, p)

        return carry

    lax.fori_loop(0, CPT, step, 0)
    pltpu.make_async_copy(out_hbm.at[pl.ds(0, CHUNK)],
                          rows_v.at[lax.rem(CPT - 2, 2)], semC).wait()
    pltpu.make_async_copy(out_hbm.at[pl.ds(0, CHUNK)],
                          rows_v.at[lax.rem(CPT - 1, 2)], semC).wait()

    plsc.subcore_barrier()

    def ocp(t, carry):
        pltpu.sync_copy(agg_sh.at[pl.ds(r0 + t * ZR, ZR)],
                        out_hbm.at[pl.ds(cid * N + r0 + t * ZR, ZR)])
        return carry

    lax.fori_loop(0, (8 * (NBLK8 // NSUB)) // ZR, ocp, 0)

    @pl.when(sid < NBLK8 % NSUB)
    def _():
        rt = r0 + 8 * nblk - 8
        pltpu.sync_copy(agg_sh.at[pl.ds(rt, 8)],
                        out_hbm.at[pl.ds(cid * N + rt, 8)])


def _sc_scatter(sna, snb, rec, senders, receivers):
    mesh = plsc.VectorSubcoreMesh(core_axis_name="c", subcore_axis_name="s",
                                  num_cores=NCORE, num_subcores=NSUB)
    f = functools.partial(
        pl.kernel,
        out_type=jax.ShapeDtypeStruct((NCORE * N, ROWW), jnp.float32),
        mesh=mesh,
        scratch_types=[
            pltpu.VMEM((2, CHUNK), jnp.int32),
            pltpu.VMEM((4, CHUNK), jnp.int32),
            pltpu.VMEM((2, CHUNK, RECW), jnp.float32),
            pltpu.VMEM((2, CHUNK, HALF), jnp.float32),
            pltpu.VMEM((2, CHUNK, ROWW), jnp.float32),
            pltpu.VMEM((ZR, ROWW), jnp.float32),
            pltpu.VMEM_SHARED((N, ROWW), jnp.float32),
            pltpu.SemaphoreType.DMA,
            pltpu.SemaphoreType.DMA,
            pltpu.SemaphoreType.DMA,
        ],
        compiler_params=pltpu.CompilerParams(use_tc_tiling_on_sc=False),
    )(_sc_body)
    return f(sna, snb, rec, senders, receivers)


def _post_body(a_ref, b_ref, w2s_ref, wp1_ref, wp2_ref, wp3_ref, wvp_ref,
               wg_ref, wr1_ref, wr2_ref, wrv_ref, rep_ref, msk_ref,
               os_ref, ov_ref, sout_ref, vf_ref):
    inv = 1.0 / AVG_NEIGH
    cat = jnp.concatenate([a_ref[...], b_ref[...]], axis=1) * inv
    m_all = jnp.dot(cat, w2s_ref[...], preferred_element_type=jnp.float32)
    s = m_all[:, 0:64]
    V = m_all[:, 64:256]
    s2 = s * s
    s_out = (jnp.dot(s, wp1_ref[...], preferred_element_type=jnp.float32)
             + jnp.dot(s2, wp2_ref[...], preferred_element_type=jnp.float32)
             + jnp.dot(s2 * s, wp3_ref[...], preferred_element_type=jnp.float32))
    g = jnp.dot(s, wg_ref[...], preferred_element_type=jnp.float32)
    gate = g * jax.nn.sigmoid(g)
    gate3 = jnp.dot(gate, rep_ref[...], preferred_element_type=jnp.float32)
    vflat = jnp.dot(V, wvp_ref[...], preferred_element_type=jnp.float32) * gate3
    hpre = jnp.dot(s_out, wr1_ref[...], preferred_element_type=jnp.float32)
    h = hpre * jax.nn.sigmoid(hpre)
    os_ref[...] = jnp.dot(h, wr2_ref[...], preferred_element_type=jnp.float32)
    vgate = jnp.dot(h, wrv_ref[...], preferred_element_type=jnp.float32)
    vg3 = jnp.dot(vgate, rep_ref[...], preferred_element_type=jnp.float32)
    prod = vflat * vg3
    msk = msk_ref[...]
    ov_ref[...] = jnp.concatenate(
        [jnp.sum(prod * msk[0:1, :], axis=1, keepdims=True),
         jnp.sum(prod * msk[1:2, :], axis=1, keepdims=True),
         jnp.sum(prod * msk[2:3, :], axis=1, keepdims=True)], axis=1)
    sout_ref[...] = s_out
    vf_ref[...] = vflat


def _post_stage(agg2, w2s, Wp1, Wp2, Wp3, wvp, Wg, Wr1, Wr2, Wrv, rep, msk):
    bn = 1000
    nblk = N // bn
    wspec = lambda shape: pl.BlockSpec(shape, lambda i: (0, 0))
    return pl.pallas_call(
        _post_body,
        grid=(nblk,),
        in_specs=[
            pl.BlockSpec((bn, ROWW), lambda i: (i, 0)),
            pl.BlockSpec((bn, ROWW), lambda i: (i + nblk, 0)),
            wspec((2 * ROWW, 256)),
            wspec((NF, NF)), wspec((NF, NF)), wspec((NF, NF)),
            wspec((192, 192)), wspec((NF, NF)),
            wspec((NF, 16)), wspec((16, NF)), wspec((16, NF)),
            wspec((NF, 192)),
            wspec((8, 192)),
        ],
        out_specs=[
            pl.BlockSpec((bn, NF), lambda i: (i, 0)),
            pl.BlockSpec((bn, 3), lambda i: (i, 0)),
            pl.BlockSpec((bn, NF), lambda i: (i, 0)),
            pl.BlockSpec((bn, 192), lambda i: (i, 0)),
        ],
        out_shape=[
            jax.ShapeDtypeStruct((N, NF), jnp.float32),
            jax.ShapeDtypeStruct((N, 3), jnp.float32),
            jax.ShapeDtypeStruct((N, NF), jnp.float32),
            jax.ShapeDtypeStruct((N, 192), jnp.float32),
        ],
    )(agg2, agg2, w2s, Wp1, Wp2, Wp3, wvp, Wg, Wr1, Wr2, Wrv, rep, msk)


def kernel(vectors, lengths, node_feats, edge_feats, edge_index,
           W1, We1, We2, W2, Wp1, Wp2, Wp3, Wv, Wg, Wr1, Wr2, Wrv):
    senders = edge_index[0]
    receivers = edge_index[1]

    sna, snb = _node_lin(node_feats, W1[:, :NF])
    we1p = jnp.pad(We1, ((0, 7), (0, 0)))
    rec = _edge_stage(edge_feats, lengths, vectors, we1p, We2)

    agg2 = _sc_scatter(sna, snb, rec, senders, receivers)

    eye4 = jnp.eye(4, dtype=W2.dtype)
    w2s = jnp.concatenate([jnp.kron(eye4, W2[:HALF]),
                           jnp.kron(eye4, W2[HALF:])], axis=0)

    pmat = np.zeros((192, 192), np.float32)
    for cc in range(3):
        for j in range(NF):
            pmat[cc * NF + j, 3 * j + cc] = 1.0
    wvp = jnp.dot(jnp.kron(jnp.eye(3, dtype=Wv.dtype), Wv), jnp.asarray(pmat))
    rep = jnp.asarray(np.kron(np.eye(NF, dtype=np.float32),
                              np.ones((1, 3), np.float32)))
    tt = np.arange(3 * NF)
    mskn = np.zeros((8, 192), np.float32)
    for cc in range(3):
        mskn[cc] = (tt % 3 == cc).astype(np.float32)
    msk = jnp.asarray(mskn)

    out_scalars, out_vec, s_out, vflat = _post_stage(
        agg2, w2s, Wp1, Wp2, Wp3, wvp, Wg, Wr1, Wr2, Wrv, rep, msk)
    node_feats_out = jnp.concatenate([s_out, vflat], axis=1)
    return (out_scalars, out_vec, node_feats_out)
